# baseline (speedup 1.0000x reference)
import functools

import jax
import jax.numpy as jnp
from jax import lax
from jax.experimental import pallas as pl
from jax.experimental.pallas import tpu as pltpu
from jax.experimental.pallas import tpu_sc as plsc

VOCAB = 1000000
EMBED_DIM = 32
BATCH = 16384
FIELDS = 100

NUM_CORES = 2
NUM_SUBCORES = 16
NUM_WORKERS = NUM_CORES * NUM_SUBCORES

TOTAL = BATCH * FIELDS
PART_ROWS = (5120, 5120, 5120, 1024)
CHUNK = 320
NBUF = 2
WIDE_ROWS = VOCAB // 4


def _emb_body(x_hbm, table_hbm, out_hbm, *scratch, per_worker):
    num_chunks = per_worker // CHUNK
    wid = lax.axis_index("s") * NUM_CORES + lax.axis_index("c")
    base = wid * per_worker
    idx = list(scratch[0:NBUF])
    widx = list(scratch[NBUF:2 * NBUF])
    wide = list(scratch[2 * NBUF:3 * NBUF])
    outw = list(scratch[3 * NBUF:4 * NBUF])
    sg = list(scratch[4 * NBUF:5 * NBUF])
    ss = list(scratch[5 * NBUF:6 * NBUF])

    def prep(g, b):
        pltpu.sync_copy(x_hbm.at[pl.ds(base + g * CHUNK, CHUNK)], idx[b])

        @pl.loop(0, CHUNK // 16)
        def _w(i):
            v = idx[b][pl.ds(i * 16, 16)]
            widx[b][pl.ds(i * 16, 16)] = lax.shift_right_logical(v, 2)

    def start_gather(b):
        pltpu.async_copy(table_hbm.at[widx[b]], wide[b], sg[b])

    def wait_gather(b):
        pltpu.make_async_copy(table_hbm.at[widx[b]], wide[b], sg[b]).wait()

    def extract(b):
        @pl.loop(0, CHUNK // 16)
        def _blk(i):
            j0 = i * 16
            sv = lax.bitwise_and(idx[b][pl.ds(j0, 16)], 3) * 32
            orow0 = lax.shift_right_logical(j0, 4) * 4
            for l in range(16):
                s = sv[l]
                orow = orow0 + l // 4
                ocol = (l % 4) * 32
                for h in range(2):
                    outw[b][orow, pl.ds(ocol + h * 16, 16)] = (
                        wide[b][j0 + l, pl.ds(s + h * 16, 16)])

    def start_store(g, b):
        off = pl.multiple_of((base + g * CHUNK) // 4, 8)
        pltpu.async_copy(outw[b], out_hbm.at[pl.ds(off, CHUNK // 4)], ss[b])

    def wait_store(g, b):
        off = pl.multiple_of((base + g * CHUNK) // 4, 8)
        pltpu.make_async_copy(outw[b], out_hbm.at[pl.ds(off, CHUNK // 4)],
                              ss[b]).wait()

    for b in range(NBUF):
        prep(b, b)
        start_gather(b)

    @pl.loop(0, num_chunks - NBUF, step=NBUF)
    def _round(g):
        for j in range(NBUF):
            b = j
            k = g + j
            wait_gather(b)

            @pl.when(k >= NBUF)
            def _():
                wait_store(lax.max(k - NBUF, 0), b)

            extract(b)
            start_store(k, b)
            prep(k + NBUF, b)
            start_gather(b)

    for j in range(NBUF):
        k = num_chunks - NBUF + j
        wait_gather(j)
        wait_store(k - NBUF, j)
        extract(j)
        start_store(k, j)
    for j in range(NBUF):
        wait_store(num_chunks - NBUF + j, j)


@functools.partial(jax.jit, static_argnames=("n",))
def _embed(x_flat, table_wide, n):
    per_worker = n // NUM_WORKERS
    mesh = plsc.VectorSubcoreMesh(core_axis_name="c", subcore_axis_name="s")
    return pl.kernel(
        functools.partial(_emb_body, per_worker=per_worker),
        out_type=jax.ShapeDtypeStruct((n // 4, 128), jnp.float32),
        mesh=mesh,
        scratch_types=(
            [pltpu.VMEM((CHUNK,), jnp.int32)] * (2 * NBUF)
            + [pltpu.VMEM((CHUNK, 128), jnp.float32)] * NBUF
            + [pltpu.VMEM((CHUNK // 4, 128), jnp.float32)] * NBUF
            + [pltpu.SemaphoreType.DMA] * (2 * NBUF)
        ),
        compiler_params=pltpu.CompilerParams(use_tc_tiling_on_sc=True,
                                             needs_layout_passes=False),
    )(x_flat, table_wide)


def kernel(x, table):
    tw = table.reshape(WIDE_ROWS, 128)
    parts = []
    r0 = 0
    for rows in PART_ROWS:
        n = rows * FIELDS
        parts.append(_embed(x[r0:r0 + rows].reshape(-1), tw, n)
                     .reshape(rows, FIELDS, EMBED_DIM))
        r0 += rows
    return jnp.concatenate(parts, axis=0)


# speedup vs baseline: 1.0384x; 1.0384x over previous
import jax
import jax.numpy as jnp
from jax import lax
from jax.experimental import pallas as pl
from jax.experimental.pallas import tpu as pltpu
from jax.experimental.pallas import tpu_sc as plsc

VOCAB = 1000000
EMBED_DIM = 32
BATCH = 16384
FIELDS = 100

NUM_CORES = 2
NUM_SUBCORES = 16
NUM_WORKERS = NUM_CORES * NUM_SUBCORES

TOTAL = BATCH * FIELDS
HALVES = 4
N_HALF = TOTAL // HALVES
PER_WORKER = N_HALF // NUM_WORKERS
CHUNK = 320
NUM_CHUNKS = PER_WORKER // CHUNK
NBUF = 2
WIDE_ROWS = VOCAB // 4


def _emb_body(x_hbm, table_hbm, out_hbm, *scratch):
    wid = lax.axis_index("s") * NUM_CORES + lax.axis_index("c")
    base = wid * PER_WORKER
    idx = list(scratch[0:NBUF])
    widx = list(scratch[NBUF:2 * NBUF])
    wide = list(scratch[2 * NBUF:3 * NBUF])
    outw = list(scratch[3 * NBUF:4 * NBUF])
    sg = list(scratch[4 * NBUF:5 * NBUF])
    ss = list(scratch[5 * NBUF:6 * NBUF])

    def prep(g, b):
        pltpu.sync_copy(x_hbm.at[pl.ds(base + g * CHUNK, CHUNK)], idx[b])

        @pl.loop(0, CHUNK // 16)
        def _w(i):
            v = idx[b][pl.ds(i * 16, 16)]
            widx[b][pl.ds(i * 16, 16)] = lax.shift_right_logical(v, 2)

    def start_gather(b):
        pltpu.async_copy(table_hbm.at[widx[b]], wide[b], sg[b])

    def wait_gather(b):
        pltpu.make_async_copy(table_hbm.at[widx[b]], wide[b], sg[b]).wait()

    def extract(b):
        @pl.loop(0, CHUNK // 16)
        def _blk(i):
            j0 = i * 16
            sv = lax.bitwise_and(idx[b][pl.ds(j0, 16)], 3) * 32
            orow0 = lax.shift_right_logical(j0, 4) * 4
            for l in range(16):
                s = sv[l]
                orow = orow0 + l // 4
                ocol = (l % 4) * 32
                for h in range(2):
                    outw[b][orow, pl.ds(ocol + h * 16, 16)] = (
                        wide[b][j0 + l, pl.ds(s + h * 16, 16)])

    def start_store(g, b):
        off = pl.multiple_of((base + g * CHUNK) // 4, 8)
        pltpu.async_copy(outw[b], out_hbm.at[pl.ds(off, CHUNK // 4)], ss[b])

    def wait_store(g, b):
        off = pl.multiple_of((base + g * CHUNK) // 4, 8)
        pltpu.make_async_copy(outw[b], out_hbm.at[pl.ds(off, CHUNK // 4)],
                              ss[b]).wait()

    for b in range(NBUF):
        prep(b, b)
        start_gather(b)

    @pl.loop(0, NUM_CHUNKS - NBUF, step=NBUF)
    def _round(g):
        for j in range(NBUF):
            b = j
            k = g + j
            wait_gather(b)

            @pl.when(k >= NBUF)
            def _():
                wait_store(lax.max(k - NBUF, 0), b)

            extract(b)
            start_store(k, b)
            prep(k + NBUF, b)
            start_gather(b)

    for j in range(NBUF):
        k = NUM_CHUNKS - NBUF + j
        wait_gather(j)
        wait_store(k - NBUF, j)
        extract(j)
        start_store(k, j)
    for j in range(NBUF):
        wait_store(NUM_CHUNKS - NBUF + j, j)


@jax.jit
def _embed(x_flat, table_wide):
    mesh = plsc.VectorSubcoreMesh(core_axis_name="c", subcore_axis_name="s")
    return pl.kernel(
        _emb_body,
        out_type=jax.ShapeDtypeStruct((N_HALF // 4, 128), jnp.float32),
        mesh=mesh,
        scratch_types=(
            [pltpu.VMEM((CHUNK,), jnp.int32)] * (2 * NBUF)
            + [pltpu.VMEM((CHUNK, 128), jnp.float32)] * NBUF
            + [pltpu.VMEM((CHUNK // 4, 128), jnp.float32)] * NBUF
            + [pltpu.SemaphoreType.DMA] * (2 * NBUF)
        ),
        compiler_params=pltpu.CompilerParams(use_tc_tiling_on_sc=True,
                                             needs_layout_passes=False),
    )(x_flat, table_wide)


def kernel(x, table):
    tw = table.reshape(WIDE_ROWS, 128)
    rows = BATCH // HALVES
    halves = [
        _embed(x[h * rows:(h + 1) * rows].reshape(-1), tw)
        .reshape(rows, FIELDS, EMBED_DIM)
        for h in range(HALVES)
    ]
    return jnp.concatenate(halves, axis=0)
